# baseline (speedup 1.0000x reference)
import functools

import jax
import jax.numpy as jnp
from jax import lax
from jax.experimental import pallas as pl
from jax.experimental.pallas import tpu as pltpu
from jax.experimental.pallas import tpu_sc as plsc

EMB = 128
NC = 2
NS = 16
NW = NC * NS

C = 64
BT = 5
BS = 4
STEP = 20


@functools.cache
def _build(n_rows: int):
    assert n_rows % (NW * C) == 0
    bpw = n_rows // NW
    nchunk = bpw // C
    ngroup = nchunk // STEP
    assert ngroup * STEP == nchunk and ngroup >= 2

    mesh = plsc.VectorSubcoreMesh(core_axis_name="c", subcore_axis_name="s")

    @functools.partial(
        pl.kernel,
        out_type=jax.ShapeDtypeStruct((n_rows, EMB), jnp.float32),
        mesh=mesh,
        scratch_types=[
            pltpu.VMEM((nchunk, C), jnp.int32),
            pltpu.VMEM((BT, C, EMB), jnp.float32),
            pltpu.VMEM_SHARED((NS, BS, C, EMB), jnp.float32),
        ]
        + [pltpu.SemaphoreType.DMA] * (BT + 2 * BS),
    )
    def emb(idx_hbm, table_hbm, out_hbm, idx_v, rows_v, sh, *sems):
        gsems = sems[:BT]
        xsems = sems[BT : BT + BS]
        osems = sems[BT + BS :]
        cid = lax.axis_index("c")
        sid = lax.axis_index("s")
        wid = sid * NC + cid
        base = wid * bpw

        pltpu.sync_copy(idx_hbm.at[wid], idx_v)

        def g_start(j, st):
            pltpu.async_copy(table_hbm.at[idx_v.at[j]], rows_v.at[st], gsems[st])

        def g_wait(st):
            pltpu.make_async_copy(
                table_hbm.at[idx_v.at[0]], rows_v.at[st], gsems[st]
            ).wait()

        def x_start(st, ss):
            pltpu.async_copy(rows_v.at[st], sh.at[sid, ss], xsems[ss])

        def x_wait(st, ss):
            pltpu.make_async_copy(
                rows_v.at[st], sh.at[sid, ss], xsems[ss]
            ).wait()

        def o_start(j, ss):
            pltpu.async_copy(
                sh.at[sid, ss], out_hbm.at[pl.ds(base + j * C, C)], osems[ss]
            )

        def o_wait(ss):
            pltpu.make_async_copy(
                sh.at[sid, ss], out_hbm.at[pl.ds(base, C)], osems[ss]
            ).wait()

        def step(j, jmax):
            if j <= jmax:
                g_start(j, j % BT)
            if 0 <= j - 2 <= jmax:
                g_wait((j - 2) % BT)
            if 0 <= j - 2 - BS <= jmax:
                o_wait((j - 2) % BS)
            if 0 <= j - 2 <= jmax:
                x_start((j - 2) % BT, (j - 2) % BS)
            if 0 <= j - 4 <= jmax:
                x_wait((j - 4) % BT, (j - 4) % BS)
                o_start(j - 4, (j - 4) % BS)

        last = nchunk - 1
        for j in range(STEP):
            step(j, last)

        @pl.loop(1, ngroup)
        def group(g):
            for b in range(STEP):
                j = g * STEP + b
                g_start(j, b % BT)
                g_wait((b - 2) % BT)
                o_wait((b - 2) % BS)
                x_start((b - 2) % BT, (b - 2) % BS)
                x_wait((b - 4) % BT, (b - 4) % BS)
                o_start(j - 4, (b - 4) % BS)

        for j in range(nchunk, nchunk + BS + 2):
            step(j, last)

    return emb


def kernel(x, table):
    n_rows = x.size
    idx = x.reshape(NW, n_rows // (NW * C), C).astype(jnp.int32)
    out = _build(n_rows)(idx, table)
    return out.reshape(x.shape + (EMB,))


# speedup vs baseline: 1.0129x; 1.0129x over previous
import functools

import jax
import jax.numpy as jnp
from jax import lax
from jax.experimental import pallas as pl
from jax.experimental.pallas import tpu as pltpu
from jax.experimental.pallas import tpu_sc as plsc

EMB = 128
NC = 2
NS = 16
NW = NC * NS

C = 64
B = 10
K = 5


@functools.cache
def _build(n_rows: int):
    assert n_rows % (NW * C) == 0
    bpw = n_rows // NW
    nchunk = bpw // C
    ngroup = nchunk // B
    assert ngroup * B == nchunk and ngroup >= 2

    mesh = plsc.VectorSubcoreMesh(core_axis_name="c", subcore_axis_name="s")

    @functools.partial(
        pl.kernel,
        out_type=jax.ShapeDtypeStruct((n_rows, EMB), jnp.float32),
        mesh=mesh,
        scratch_types=[
            pltpu.VMEM((nchunk, C), jnp.int32),
            pltpu.VMEM((B, C, EMB), jnp.float32),
        ]
        + [pltpu.SemaphoreType.DMA] * (2 * B),
    )
    def emb(idx_hbm, table_hbm, out_hbm, idx_v, rows_v, *sems):
        gsems = sems[:B]
        osems = sems[B:]
        wid = lax.axis_index("s") * NC + lax.axis_index("c")
        base = wid * bpw

        pltpu.sync_copy(idx_hbm.at[wid], idx_v)

        def g_start(j, s):
            pltpu.async_copy(table_hbm.at[idx_v.at[j]], rows_v.at[s], gsems[s])

        def g_wait(s):
            pltpu.make_async_copy(
                table_hbm.at[idx_v.at[0]], rows_v.at[s], gsems[s]
            ).wait()

        def o_start(j, s):
            pltpu.async_copy(
                rows_v.at[s], out_hbm.at[pl.ds(base + j * C, C)], osems[s]
            )

        def o_wait(s):
            pltpu.make_async_copy(
                rows_v.at[s], out_hbm.at[pl.ds(base, C)], osems[s]
            ).wait()

        for j in range(B):
            g_start(j, j)
            if j >= K:
                g_wait(j - K)
                o_start(j - K, j - K)

        @pl.loop(1, ngroup)
        def group(g):
            for b in range(B):
                j = g * B + b
                o_wait(b)
                g_start(j, b)
                s2 = (b - K) % B
                g_wait(s2)
                o_start(j - K, s2)

        for t in range(K):
            j = nchunk - K + t
            s = j % B
            g_wait(s)
            o_start(j, s)
        for s in range(B):
            o_wait(s)

    return emb


def kernel(x, table):
    n_rows = x.size
    idx = x.reshape(NW, n_rows // (NW * C), C).astype(jnp.int32)
    out = _build(n_rows)(idx, table)
    return out.reshape(x.shape + (EMB,))
